# baseline (speedup 1.0000x reference)
import functools

import jax
import jax.numpy as jnp
from jax import lax
from jax.experimental import pallas as pl
from jax.experimental.pallas import tpu as pltpu
from jax.experimental.pallas import tpu_sc as plsc

N = 10000
E = 160000
B = 4
C_IN = 4
T = 12
F = 64
TB = T * B
HALF = 96
WHALF = 128

CHUNK = 128
NCHUNK = 79
TILE_E = CHUNK * NCHUNK
EPAD = 16 * TILE_E
NPAD = 10240
ROWS_PER_TILE = NPAD // 16
DEGPAD = NPAD


def _prop_body(src_h, dst_h, w_h, z_h, scale_h, seg_h, deg_h,
               src_v, dst_v, w_v, nrm_v, rows_v, zb_v, zdeg_v, scale_v,
               acc_sh, deg_sh, sem):
    cid = lax.axis_index("c")
    sid = lax.axis_index("s")
    zero16 = jnp.zeros((16,), jnp.float32)

    def zrow(r, _):
        for j in range(WHALF // 16):
            zb_v[r, pl.ds(j * 16, 16)] = zero16
        return 0
    lax.fori_loop(0, CHUNK, zrow, 0)

    def zdeg(i, _):
        zdeg_v[pl.ds(i * 16, 16)] = zero16
        return 0
    lax.fori_loop(0, 40, zdeg, 0)

    for k in range(ROWS_PER_TILE // CHUNK):
        pltpu.sync_copy(zb_v, acc_sh.at[pl.ds(sid * ROWS_PER_TILE + k * CHUNK, CHUNK)])

    @pl.when(cid == 0)
    def _():
        pltpu.sync_copy(zdeg_v, deg_sh.at[pl.ds(sid * 640, 640)])

    pltpu.sync_copy(scale_h, scale_v)
    svec = scale_v[...]
    plsc.subcore_barrier()

    base_e = sid * TILE_E

    def chunk(k, _):
        eb = base_e + k * CHUNK
        pltpu.sync_copy(src_h.at[pl.ds(eb, CHUNK)], src_v)
        pltpu.sync_copy(dst_h.at[pl.ds(eb, CHUNK)], dst_v)
        pltpu.sync_copy(w_h.at[pl.ds(eb, CHUNK)], w_v)
        pltpu.async_copy(z_h.at[cid].at[src_v], rows_v, sem).wait()

        def g16(g, _):
            s = src_v[pl.ds(g * 16, 16)]
            d = dst_v[pl.ds(g * 16, 16)]
            wv = w_v[pl.ds(g * 16, 16)]
            we = jnp.where(s == d, jnp.zeros((16,), jnp.float32), wv)
            w_v[pl.ds(g * 16, 16)] = we
            nrm_v[pl.ds(g * 16, 16)] = we * svec
            return 0
        lax.fori_loop(0, CHUNK // 16, g16, 0)

        @pl.when(cid == 0)
        def _():
            pltpu.sync_copy(w_v, deg_sh.at[src_v], add=True)

        def scale_rows(g, _):
            nrm16 = nrm_v[pl.ds(g * 16, 16)]
            for j in range(16):
                e = g * 16 + j
                bj = jax.lax.broadcast_in_dim(
                    jax.lax.slice(nrm16, (j,), (j + 1,)), (16,), (0,))
                for f in range(HALF // 16):
                    rows_v[e, pl.ds(f * 16, 16)] = rows_v[e, pl.ds(f * 16, 16)] * bj
            return 0
        lax.fori_loop(0, CHUNK // 16, scale_rows, 0)

        pltpu.sync_copy(rows_v, acc_sh.at[dst_v], add=True)
        return 0

    lax.fori_loop(0, NCHUNK, chunk, 0)
    plsc.subcore_barrier()

    r0 = sid * ROWS_PER_TILE
    pltpu.sync_copy(acc_sh.at[pl.ds(r0, ROWS_PER_TILE)],
                    seg_h.at[cid].at[pl.ds(r0, ROWS_PER_TILE)])

    @pl.when(cid == 0)
    def _():
        pltpu.sync_copy(deg_sh.at[pl.ds(sid * 640, 640)],
                        deg_h.at[pl.ds(sid * 640, 640)])


@functools.lru_cache(maxsize=None)
def _make_prop_sc():
  return pl.kernel(
    _prop_body,
    out_type=(jax.ShapeDtypeStruct((2, NPAD, WHALF), jnp.float32),
              jax.ShapeDtypeStruct((DEGPAD,), jnp.float32)),
    mesh=plsc.VectorSubcoreMesh(core_axis_name="c", subcore_axis_name="s",
                                num_cores=2, num_subcores=16),
    scratch_types=[
        pltpu.VMEM((CHUNK,), jnp.int32),
        pltpu.VMEM((CHUNK,), jnp.int32),
        pltpu.VMEM((CHUNK,), jnp.float32),
        pltpu.VMEM((CHUNK,), jnp.float32),
        pltpu.VMEM((CHUNK, WHALF), jnp.float32),
        pltpu.VMEM((CHUNK, WHALF), jnp.float32),
        pltpu.VMEM((640,), jnp.float32),
        pltpu.VMEM((16,), jnp.float32),
        pltpu.VMEM_SHARED((NPAD, WHALF), jnp.float32),
        pltpu.VMEM_SHARED((DEGPAD,), jnp.float32),
        pltpu.SemaphoreType.DMA,
    ],
    name="cheb_prop_sc",
  )


def _combine_body(lam_ref, seg_ref, zp_ref, zpp_ref, deg_ref, o_ref, *, alpha, beta):
    lam = lam_ref[0, 0]
    diag = 2.0 * deg_ref[...] / lam - 1.0
    o_ref[...] = (alpha * (seg_ref[...] + diag * zp_ref[...])
                  + beta * zpp_ref[...])


def _combine(lam_arr, seg2, zp2, zpp2, deg2, alpha, beta):
    nb = 1024
    grid = (2, NPAD // nb)
    return pl.pallas_call(
        functools.partial(_combine_body, alpha=alpha, beta=beta),
        grid=grid,
        in_specs=[
            pl.BlockSpec((1, 1), lambda h, i: (0, 0)),
            pl.BlockSpec((nb, WHALF), lambda h, i: (h * (NPAD // nb) + i, 0)),
            pl.BlockSpec((nb, WHALF), lambda h, i: (h * (NPAD // nb) + i, 0)),
            pl.BlockSpec((nb, WHALF), lambda h, i: (h * (NPAD // nb) + i, 0)),
            pl.BlockSpec((nb, 1), lambda h, i: (i, 0)),
        ],
        out_specs=pl.BlockSpec((nb, WHALF), lambda h, i: (h * (NPAD // nb) + i, 0)),
        out_shape=jax.ShapeDtypeStruct((2 * NPAD, WHALF), jnp.float32),
        name="cheb_combine",
    )(lam_arr, seg2, zp2, zpp2, deg2)


def _dense_body(z0_ref, z1_ref, z2_ref, x_ref, w0_ref, w1_ref, w2_ref,
                cb_ref, wt_ref, tb_ref, wr_ref, rb_ref, lg_ref, lb_ref,
                o_ref, *, nb):
    rows = nb * TB
    dot = functools.partial(jnp.dot, preferred_element_type=jnp.float32)
    r = (dot(z0_ref[...], w0_ref[...]) + dot(z1_ref[...], w1_ref[...])
         + dot(z2_ref[...], w2_ref[...]) + cb_ref[...])
    r = jnp.maximum(r, 0.0)
    rv = r.reshape(nb * B, T, F)
    zpad = jnp.zeros((nb * B, 1, F), jnp.float32)
    rp = jnp.concatenate([zpad, rv, zpad], axis=1)
    h = (dot(rp[:, 0:T, :].reshape(rows, F), wt_ref[0])
         + dot(rp[:, 1:T + 1, :].reshape(rows, F), wt_ref[1])
         + dot(rp[:, 2:T + 2, :].reshape(rows, F), wt_ref[2])
         + tb_ref[...])
    res = dot(x_ref[...], wr_ref[...]) + rb_ref[...]
    z = jnp.maximum(h + res, 0.0)
    mu = jnp.mean(z, axis=-1, keepdims=True)
    zc = z - mu
    var = jnp.mean(zc * zc, axis=-1, keepdims=True)
    z = zc * jax.lax.rsqrt(var + 1e-5) * lg_ref[...] + lb_ref[...]
    zr = z.reshape(nb, B, T, F)
    for b in range(B):
        o_ref[b] = jnp.swapaxes(zr[:, b], 1, 2)


def _dense(z0r, z1r, z2r, xrr, w0, w1, w2, cb, wt, tb, wr, rb, lg, lb):
    nb = 50
    rows = nb * TB
    grid = (N // nb,)
    row_spec = pl.BlockSpec((rows, C_IN), lambda i: (i, 0))
    full = lambda s: pl.BlockSpec(s, lambda i: tuple(0 for _ in s))
    return pl.pallas_call(
        functools.partial(_dense_body, nb=nb),
        grid=grid,
        in_specs=[row_spec, row_spec, row_spec, row_spec,
                  full((C_IN, F)), full((C_IN, F)), full((C_IN, F)),
                  full((1, F)), full((3, F, F)), full((1, F)),
                  full((C_IN, F)), full((1, F)), full((1, F)), full((1, F))],
        out_specs=pl.BlockSpec((B, nb, F, T), lambda i: (0, i, 0, 0)),
        out_shape=jax.ShapeDtypeStruct((B, N, F, T), jnp.float32),
        name="mstgcn_dense",
    )(z0r, z1r, z2r, xrr, w0, w1, w2, cb, wt, tb, wr, rb, lg, lb)


def kernel(X, edge_index, edge_weight, lambda_max, cheb_W, cheb_b,
           time_W, time_b, res_W, res_b, ln_g, ln_b):
    lam = jnp.asarray(lambda_max, jnp.float32)
    src = edge_index[0]
    dst = edge_index[1]
    pad = EPAD - E
    srcp = jnp.concatenate([src, jnp.zeros((pad,), jnp.int32)])
    dstp = jnp.concatenate([dst, jnp.zeros((pad,), jnp.int32)])
    wp = jnp.concatenate([edge_weight, jnp.zeros((pad,), jnp.float32)])
    scale_arr = jnp.full((16,), -2.0, jnp.float32) / lam

    Xt = jnp.transpose(X, (2, 0, 1, 3)).reshape(N, C_IN, TB)
    Xt = jnp.transpose(Xt, (2, 0, 1))
    xr = jnp.transpose(Xt, (1, 0, 2)).reshape(N, TB * C_IN)

    def to_split(z):
        zs = jnp.transpose(z.reshape(N, 2, HALF), (1, 0, 2))
        return jnp.pad(zs, ((0, 0), (0, NPAD - N), (0, WHALF - HALF)))

    def from_split2(z2):
        zs = z2.reshape(2, NPAD, WHALF)[:, :N, :HALF]
        return jnp.transpose(zs, (1, 0, 2)).reshape(N, 2 * HALF)

    _prop_sc = _make_prop_sc()
    z0s = to_split(xr)
    seg1, degp = _prop_sc(srcp, dstp, wp, z0s, scale_arr)
    deg2 = degp.reshape(NPAD, 1)
    lam_arr = lam.reshape(1, 1)

    z0f = z0s.reshape(2 * NPAD, WHALF)
    z1f = _combine(lam_arr, seg1.reshape(2 * NPAD, WHALF), z0f, z0f,
                   deg2, 1.0, 0.0)

    seg2, _ = _prop_sc(srcp, dstp, wp, z1f.reshape(2, NPAD, WHALF), scale_arr)
    z2f = _combine(lam_arr, seg2.reshape(2 * NPAD, WHALF), z1f, z0f,
                   deg2, 2.0, -1.0)

    z0r = xr.reshape(N * TB, C_IN)
    z1r = from_split2(z1f).reshape(N * TB, C_IN)
    z2r = from_split2(z2f).reshape(N * TB, C_IN)
    xrr = jnp.transpose(X, (1, 0, 3, 2)).reshape(N * B * T, C_IN)

    w0, w1, w2 = cheb_W[0], cheb_W[1], cheb_W[2]
    cb = cheb_b.reshape(1, F)
    wt = jnp.transpose(time_W[:, :, 0, :], (2, 1, 0))
    tb = time_b.reshape(1, F)
    wr = jnp.transpose(res_W[:, :, 0, 0])
    rb = res_b.reshape(1, F)

    return _dense(z0r, z1r, z2r, xrr, w0, w1, w2, cb, wt, tb, wr, rb,
                  ln_g.reshape(1, F), ln_b.reshape(1, F))


# speedup vs baseline: 1.4872x; 1.4872x over previous
import functools

import jax
import jax.numpy as jnp
from jax import lax
from jax.experimental import pallas as pl
from jax.experimental.pallas import tpu as pltpu
from jax.experimental.pallas import tpu_sc as plsc

N = 10000
E = 160000
B = 4
C_IN = 4
T = 12
F = 64
TB = T * B
HALF = 96
WHALF = 128

CHUNK = 128
NCHUNK = 80
TILE_E = CHUNK * NCHUNK
EPAD = 16 * TILE_E
NPAD = 10240
ROWS_PER_TILE = NPAD // 16
DEGPAD = NPAD


def _prop_body(src_h, dst_h, w_h, z_h, scale_h, seg_h, deg_h,
               src_a, dst_a, w_a, nrm_a, rows_a,
               src_b, dst_b, w_b, nrm_b, rows_b,
               zb_v, zdeg_v, scale_v, acc_sh, deg_sh, sem_a, sem_b):
    cid = lax.axis_index("c")
    sid = lax.axis_index("s")
    zero16 = jnp.zeros((16,), jnp.float32)

    def zrow(r, _):
        for j in range(WHALF // 16):
            zb_v[r, pl.ds(j * 16, 16)] = zero16
        return 0
    lax.fori_loop(0, 32, zrow, 0)

    def zdeg(i, _):
        zdeg_v[pl.ds(i * 16, 16)] = zero16
        return 0
    lax.fori_loop(0, 40, zdeg, 0)

    for k in range(ROWS_PER_TILE // 32):
        pltpu.sync_copy(zb_v, acc_sh.at[pl.ds(sid * ROWS_PER_TILE + k * 32, 32)])

    @pl.when(cid == 0)
    def _():
        pltpu.sync_copy(zdeg_v, deg_sh.at[pl.ds(sid * 640, 640)])

    pltpu.sync_copy(scale_h, scale_v)
    svec = scale_v[...]
    plsc.subcore_barrier()

    base_e = sid * TILE_E

    def load_idx(eb, src_v, dst_v, w_v):
        pltpu.sync_copy(src_h.at[pl.ds(eb, CHUNK)], src_v)
        pltpu.sync_copy(dst_h.at[pl.ds(eb, CHUNK)], dst_v)
        pltpu.sync_copy(w_h.at[pl.ds(eb, CHUNK)], w_v)

    def start_gather(src_v, rows_v, sem):
        pltpu.async_copy(z_h.at[cid].at[src_v], rows_v, sem)

    def wait_gather(src_v, rows_v, sem):
        pltpu.make_async_copy(z_h.at[cid].at[src_v], rows_v, sem).wait()

    def prep_nrm(src_v, dst_v, w_v, nrm_v):
        def g16(g, _):
            s = src_v[pl.ds(g * 16, 16)]
            d = dst_v[pl.ds(g * 16, 16)]
            wv = w_v[pl.ds(g * 16, 16)]
            we = jnp.where(s == d, jnp.zeros((16,), jnp.float32), wv)
            w_v[pl.ds(g * 16, 16)] = we
            nrm_v[pl.ds(g * 16, 16)] = we * svec
            return 0
        lax.fori_loop(0, CHUNK // 16, g16, 0)

        @pl.when(cid == 0)
        def _():
            pltpu.sync_copy(w_v, deg_sh.at[src_v], add=True)

    def scale_scatter(nrm_v, rows_v, dst_v):
        def scale_rows(g, _):
            nrm16 = nrm_v[pl.ds(g * 16, 16)]
            for j in range(16):
                e = g * 16 + j
                bj = jax.lax.broadcast_in_dim(
                    jax.lax.slice(nrm16, (j,), (j + 1,)), (16,), (0,))
                for f in range(HALF // 16):
                    rows_v[e, pl.ds(f * 16, 16)] = rows_v[e, pl.ds(f * 16, 16)] * bj
            return 0
        lax.fori_loop(0, CHUNK // 16, scale_rows, 0)
        pltpu.sync_copy(rows_v, acc_sh.at[dst_v], add=True)

    load_idx(base_e, src_a, dst_a, w_a)
    start_gather(src_a, rows_a, sem_a)

    def pair(i, _):
        prep_nrm(src_a, dst_a, w_a, nrm_a)
        load_idx(base_e + (2 * i + 1) * CHUNK, src_b, dst_b, w_b)
        start_gather(src_b, rows_b, sem_b)
        wait_gather(src_a, rows_a, sem_a)
        scale_scatter(nrm_a, rows_a, dst_a)
        prep_nrm(src_b, dst_b, w_b, nrm_b)

        @pl.when(i < NCHUNK // 2 - 1)
        def _():
            load_idx(base_e + (2 * i + 2) * CHUNK, src_a, dst_a, w_a)
            start_gather(src_a, rows_a, sem_a)

        wait_gather(src_b, rows_b, sem_b)
        scale_scatter(nrm_b, rows_b, dst_b)
        return 0

    lax.fori_loop(0, NCHUNK // 2, pair, 0)
    plsc.subcore_barrier()

    r0 = sid * ROWS_PER_TILE
    pltpu.sync_copy(acc_sh.at[pl.ds(r0, ROWS_PER_TILE)],
                    seg_h.at[cid].at[pl.ds(r0, ROWS_PER_TILE)])

    @pl.when(cid == 0)
    def _():
        pltpu.sync_copy(deg_sh.at[pl.ds(sid * 640, 640)],
                        deg_h.at[pl.ds(sid * 640, 640)])


@functools.lru_cache(maxsize=None)
def _make_prop_sc():
  return pl.kernel(
    _prop_body,
    out_type=(jax.ShapeDtypeStruct((2, NPAD, WHALF), jnp.float32),
              jax.ShapeDtypeStruct((DEGPAD,), jnp.float32)),
    mesh=plsc.VectorSubcoreMesh(core_axis_name="c", subcore_axis_name="s",
                                num_cores=2, num_subcores=16),
    scratch_types=[
        pltpu.VMEM((CHUNK,), jnp.int32),
        pltpu.VMEM((CHUNK,), jnp.int32),
        pltpu.VMEM((CHUNK,), jnp.float32),
        pltpu.VMEM((CHUNK,), jnp.float32),
        pltpu.VMEM((CHUNK, WHALF), jnp.float32),
        pltpu.VMEM((CHUNK,), jnp.int32),
        pltpu.VMEM((CHUNK,), jnp.int32),
        pltpu.VMEM((CHUNK,), jnp.float32),
        pltpu.VMEM((CHUNK,), jnp.float32),
        pltpu.VMEM((CHUNK, WHALF), jnp.float32),
        pltpu.VMEM((32, WHALF), jnp.float32),
        pltpu.VMEM((640,), jnp.float32),
        pltpu.VMEM((16,), jnp.float32),
        pltpu.VMEM_SHARED((NPAD, WHALF), jnp.float32),
        pltpu.VMEM_SHARED((DEGPAD,), jnp.float32),
        pltpu.SemaphoreType.DMA,
        pltpu.SemaphoreType.DMA,
    ],
    name="cheb_prop_sc",
  )


def _combine_body(lam_ref, seg_ref, zp_ref, zpp_ref, deg_ref, o_ref, *, alpha, beta):
    lam = lam_ref[0, 0]
    diag = 2.0 * deg_ref[...] / lam - 1.0
    o_ref[...] = (alpha * (seg_ref[...] + diag * zp_ref[...])
                  + beta * zpp_ref[...])


def _combine(lam_arr, seg2, zp2, zpp2, deg2, alpha, beta):
    nb = 1024
    grid = (2, NPAD // nb)
    return pl.pallas_call(
        functools.partial(_combine_body, alpha=alpha, beta=beta),
        grid=grid,
        in_specs=[
            pl.BlockSpec((1, 1), lambda h, i: (0, 0)),
            pl.BlockSpec((nb, WHALF), lambda h, i: (h * (NPAD // nb) + i, 0)),
            pl.BlockSpec((nb, WHALF), lambda h, i: (h * (NPAD // nb) + i, 0)),
            pl.BlockSpec((nb, WHALF), lambda h, i: (h * (NPAD // nb) + i, 0)),
            pl.BlockSpec((nb, 1), lambda h, i: (i, 0)),
        ],
        out_specs=pl.BlockSpec((nb, WHALF), lambda h, i: (h * (NPAD // nb) + i, 0)),
        out_shape=jax.ShapeDtypeStruct((2 * NPAD, WHALF), jnp.float32),
        name="cheb_combine",
    )(lam_arr, seg2, zp2, zpp2, deg2)


def _dense_body(z0_ref, z1_ref, z2_ref, x_ref, w0_ref, w1_ref, w2_ref,
                cb_ref, wt_ref, tb_ref, wr_ref, rb_ref, lg_ref, lb_ref,
                o_ref, *, nb):
    rows = nb * TB
    dot = functools.partial(jnp.dot, preferred_element_type=jnp.float32)
    r = (dot(z0_ref[...], w0_ref[...]) + dot(z1_ref[...], w1_ref[...])
         + dot(z2_ref[...], w2_ref[...]) + cb_ref[...])
    r = jnp.maximum(r, 0.0)
    rv = r.reshape(nb * B, T, F)
    zpad = jnp.zeros((nb * B, 1, F), jnp.float32)
    rp = jnp.concatenate([zpad, rv, zpad], axis=1)
    h = (dot(rp[:, 0:T, :].reshape(rows, F), wt_ref[0])
         + dot(rp[:, 1:T + 1, :].reshape(rows, F), wt_ref[1])
         + dot(rp[:, 2:T + 2, :].reshape(rows, F), wt_ref[2])
         + tb_ref[...])
    res = dot(x_ref[...], wr_ref[...]) + rb_ref[...]
    z = jnp.maximum(h + res, 0.0)
    mu = jnp.mean(z, axis=-1, keepdims=True)
    zc = z - mu
    var = jnp.mean(zc * zc, axis=-1, keepdims=True)
    o_ref[...] = zc * jax.lax.rsqrt(var + 1e-5) * lg_ref[...] + lb_ref[...]


def _dense(z0r, z1r, z2r, xrr, w0, w1, w2, cb, wt, tb, wr, rb, lg, lb):
    nb = 50
    rows = nb * TB
    grid = (N // nb,)
    row_spec = pl.BlockSpec((rows, C_IN), lambda i: (i, 0))
    full = lambda s: pl.BlockSpec(s, lambda i: tuple(0 for _ in s))
    return pl.pallas_call(
        functools.partial(_dense_body, nb=nb),
        grid=grid,
        in_specs=[row_spec, row_spec, row_spec, row_spec,
                  full((C_IN, F)), full((C_IN, F)), full((C_IN, F)),
                  full((1, F)), full((3, F, F)), full((1, F)),
                  full((C_IN, F)), full((1, F)), full((1, F)), full((1, F))],
        out_specs=pl.BlockSpec((rows, F), lambda i: (i, 0)),
        out_shape=jax.ShapeDtypeStruct((N * TB, F), jnp.float32),
        name="mstgcn_dense",
    )(z0r, z1r, z2r, xrr, w0, w1, w2, cb, wt, tb, wr, rb, lg, lb)


def kernel(X, edge_index, edge_weight, lambda_max, cheb_W, cheb_b,
           time_W, time_b, res_W, res_b, ln_g, ln_b):
    lam = jnp.asarray(lambda_max, jnp.float32)
    src = edge_index[0]
    dst = edge_index[1]
    pad = EPAD - E
    srcp = jnp.concatenate([src, jnp.zeros((pad,), jnp.int32)])
    dstp = jnp.concatenate([dst, jnp.zeros((pad,), jnp.int32)])
    wp = jnp.concatenate([edge_weight, jnp.zeros((pad,), jnp.float32)])
    scale_arr = jnp.full((16,), -2.0, jnp.float32) / lam

    Xt = jnp.transpose(X, (2, 0, 1, 3)).reshape(N, C_IN, TB)
    Xt = jnp.transpose(Xt, (2, 0, 1))
    xr = jnp.transpose(Xt, (1, 0, 2)).reshape(N, TB * C_IN)

    def to_split(z):
        zs = jnp.transpose(z.reshape(N, 2, HALF), (1, 0, 2))
        return jnp.pad(zs, ((0, 0), (0, NPAD - N), (0, WHALF - HALF)))

    def from_split2(z2):
        zs = z2.reshape(2, NPAD, WHALF)[:, :N, :HALF]
        return jnp.transpose(zs, (1, 0, 2)).reshape(N, 2 * HALF)

    _prop_sc = _make_prop_sc()
    z0s = to_split(xr)
    seg1, degp = _prop_sc(srcp, dstp, wp, z0s, scale_arr)
    deg2 = degp.reshape(NPAD, 1)
    lam_arr = lam.reshape(1, 1)

    z0f = z0s.reshape(2 * NPAD, WHALF)
    z1f = _combine(lam_arr, seg1.reshape(2 * NPAD, WHALF), z0f, z0f,
                   deg2, 1.0, 0.0)

    seg2, _ = _prop_sc(srcp, dstp, wp, z1f.reshape(2, NPAD, WHALF), scale_arr)
    z2f = _combine(lam_arr, seg2.reshape(2 * NPAD, WHALF), z1f, z0f,
                   deg2, 2.0, -1.0)

    z0r = xr.reshape(N * TB, C_IN)
    z1r = from_split2(z1f).reshape(N * TB, C_IN)
    z2r = from_split2(z2f).reshape(N * TB, C_IN)
    xrr = jnp.transpose(X, (1, 0, 3, 2)).reshape(N * B * T, C_IN)

    w0, w1, w2 = cheb_W[0], cheb_W[1], cheb_W[2]
    cb = cheb_b.reshape(1, F)
    wt = jnp.transpose(time_W[:, :, 0, :], (2, 1, 0))
    tb = time_b.reshape(1, F)
    wr = jnp.transpose(res_W[:, :, 0, 0])
    rb = res_b.reshape(1, F)

    zout = _dense(z0r, z1r, z2r, xrr, w0, w1, w2, cb, wt, tb, wr, rb,
                  ln_g.reshape(1, F), ln_b.reshape(1, F))
    return jnp.transpose(zout.reshape(N, B, T, F), (1, 0, 3, 2))


# speedup vs baseline: 1.4884x; 1.0008x over previous
import functools

import jax
import jax.numpy as jnp
from jax import lax
from jax.experimental import pallas as pl
from jax.experimental.pallas import tpu as pltpu
from jax.experimental.pallas import tpu_sc as plsc

N = 10000
E = 160000
B = 4
C_IN = 4
T = 12
F = 64
TB = T * B
HALF = 96
WHALF = 128

CHUNK = 128
NCHUNK = 80
TILE_E = CHUNK * NCHUNK
EPAD = 16 * TILE_E
NPAD = 10240
ROWS_PER_TILE = NPAD // 16
DEGPAD = NPAD


def _prop_body(src_h, dst_h, w_h, z_h, scale_h, seg_h, deg_h,
               src_a, dst_a, w_a, nrm_a, rows_a,
               src_b, dst_b, w_b, nrm_b, rows_b,
               zb_v, zdeg_v, scale_v, acc_sh, deg_sh, sem_a, sem_b, sem_d,
               *, deg_on):
    cid = lax.axis_index("c")
    sid = lax.axis_index("s")
    zero16 = jnp.zeros((16,), jnp.float32)

    def zrow(r, _):
        for j in range(WHALF // 16):
            zb_v[r, pl.ds(j * 16, 16)] = zero16
        return 0
    lax.fori_loop(0, 32, zrow, 0)

    if deg_on:
        def zdeg(i, _):
            zdeg_v[pl.ds(i * 16, 16)] = zero16
            return 0
        lax.fori_loop(0, 40, zdeg, 0)

    for k in range(ROWS_PER_TILE // 32):
        pltpu.sync_copy(zb_v, acc_sh.at[pl.ds(sid * ROWS_PER_TILE + k * 32, 32)])

    if deg_on:
        @pl.when(cid == 0)
        def _():
            pltpu.sync_copy(zdeg_v, deg_sh.at[pl.ds(sid * 640, 640)])

    pltpu.sync_copy(scale_h, scale_v)
    svec = scale_v[...]
    plsc.subcore_barrier()

    base_e = sid * TILE_E

    def load_idx(eb, src_v, dst_v, w_v):
        pltpu.sync_copy(src_h.at[pl.ds(eb, CHUNK)], src_v)
        pltpu.sync_copy(dst_h.at[pl.ds(eb, CHUNK)], dst_v)
        pltpu.sync_copy(w_h.at[pl.ds(eb, CHUNK)], w_v)

    def start_gather(src_v, rows_v, sem):
        pltpu.async_copy(z_h.at[cid].at[src_v], rows_v, sem)

    def wait_gather(src_v, rows_v, sem):
        pltpu.make_async_copy(z_h.at[cid].at[src_v], rows_v, sem).wait()

    def prep_nrm(src_v, dst_v, w_v, nrm_v):
        def g16(g, _):
            s = src_v[pl.ds(g * 16, 16)]
            d = dst_v[pl.ds(g * 16, 16)]
            wv = w_v[pl.ds(g * 16, 16)]
            we = jnp.where(s == d, jnp.zeros((16,), jnp.float32), wv)
            w_v[pl.ds(g * 16, 16)] = we
            nrm_v[pl.ds(g * 16, 16)] = we * svec
            return 0
        lax.fori_loop(0, CHUNK // 16, g16, 0)

    def start_deg(src_v, w_v):
        if deg_on:
            @pl.when(cid == 0)
            def _():
                pltpu.async_copy(w_v, deg_sh.at[src_v], sem_d, add=True)

    def wait_deg(src_v, w_v, cond):
        if deg_on:
            @pl.when(jnp.logical_and(cid == 0, cond))
            def _():
                pltpu.make_async_copy(w_v, deg_sh.at[src_v], sem_d).wait()

    def scale_scatter(nrm_v, rows_v, dst_v):
        def scale_rows(g, _):
            nrm16 = nrm_v[pl.ds(g * 16, 16)]
            for j in range(16):
                e = g * 16 + j
                bj = jax.lax.broadcast_in_dim(
                    jax.lax.slice(nrm16, (j,), (j + 1,)), (16,), (0,))
                for f in range(HALF // 16):
                    rows_v[e, pl.ds(f * 16, 16)] = rows_v[e, pl.ds(f * 16, 16)] * bj
            return 0
        lax.fori_loop(0, CHUNK // 16, scale_rows, 0)
        pltpu.sync_copy(rows_v, acc_sh.at[dst_v], add=True)

    load_idx(base_e, src_a, dst_a, w_a)
    start_gather(src_a, rows_a, sem_a)

    def pair(i, _):
        prep_nrm(src_a, dst_a, w_a, nrm_a)
        start_deg(src_a, w_a)
        wait_deg(src_b, w_b, i > 0)
        load_idx(base_e + (2 * i + 1) * CHUNK, src_b, dst_b, w_b)
        start_gather(src_b, rows_b, sem_b)
        wait_gather(src_a, rows_a, sem_a)
        scale_scatter(nrm_a, rows_a, dst_a)
        prep_nrm(src_b, dst_b, w_b, nrm_b)
        start_deg(src_b, w_b)

        @pl.when(i < NCHUNK // 2 - 1)
        def _():
            wait_deg(src_a, w_a, True)
            load_idx(base_e + (2 * i + 2) * CHUNK, src_a, dst_a, w_a)
            start_gather(src_a, rows_a, sem_a)

        wait_gather(src_b, rows_b, sem_b)
        scale_scatter(nrm_b, rows_b, dst_b)
        return 0

    lax.fori_loop(0, NCHUNK // 2, pair, 0)
    wait_deg(src_a, w_a, True)
    wait_deg(src_b, w_b, True)
    plsc.subcore_barrier()

    r0 = sid * ROWS_PER_TILE
    pltpu.sync_copy(acc_sh.at[pl.ds(r0, ROWS_PER_TILE)],
                    seg_h.at[cid].at[pl.ds(r0, ROWS_PER_TILE)])

    if deg_on:
        @pl.when(cid == 0)
        def _():
            pltpu.sync_copy(deg_sh.at[pl.ds(sid * 640, 640)],
                            deg_h.at[pl.ds(sid * 640, 640)])


@functools.lru_cache(maxsize=None)
def _make_prop_sc(deg_on=True):
  return pl.kernel(
    functools.partial(_prop_body, deg_on=deg_on),
    out_type=(jax.ShapeDtypeStruct((2, NPAD, WHALF), jnp.float32),
              jax.ShapeDtypeStruct((DEGPAD,), jnp.float32)),
    mesh=plsc.VectorSubcoreMesh(core_axis_name="c", subcore_axis_name="s",
                                num_cores=2, num_subcores=16),
    scratch_types=[
        pltpu.VMEM((CHUNK,), jnp.int32),
        pltpu.VMEM((CHUNK,), jnp.int32),
        pltpu.VMEM((CHUNK,), jnp.float32),
        pltpu.VMEM((CHUNK,), jnp.float32),
        pltpu.VMEM((CHUNK, WHALF), jnp.float32),
        pltpu.VMEM((CHUNK,), jnp.int32),
        pltpu.VMEM((CHUNK,), jnp.int32),
        pltpu.VMEM((CHUNK,), jnp.float32),
        pltpu.VMEM((CHUNK,), jnp.float32),
        pltpu.VMEM((CHUNK, WHALF), jnp.float32),
        pltpu.VMEM((32, WHALF), jnp.float32),
        pltpu.VMEM((640,), jnp.float32),
        pltpu.VMEM((16,), jnp.float32),
        pltpu.VMEM_SHARED((NPAD, WHALF), jnp.float32),
        pltpu.VMEM_SHARED((DEGPAD,), jnp.float32),
        pltpu.SemaphoreType.DMA,
        pltpu.SemaphoreType.DMA,
        pltpu.SemaphoreType.DMA,
    ],
    name="cheb_prop_sc",
  )


def _combine_body(lam_ref, seg_ref, zp_ref, zpp_ref, deg_ref, o_ref, *, alpha, beta):
    lam = lam_ref[0, 0]
    diag = 2.0 * deg_ref[...] / lam - 1.0
    o_ref[...] = (alpha * (seg_ref[...] + diag * zp_ref[...])
                  + beta * zpp_ref[...])


def _combine(lam_arr, seg2, zp2, zpp2, deg2, alpha, beta):
    nb = 1024
    grid = (2, NPAD // nb)
    return pl.pallas_call(
        functools.partial(_combine_body, alpha=alpha, beta=beta),
        grid=grid,
        in_specs=[
            pl.BlockSpec((1, 1), lambda h, i: (0, 0)),
            pl.BlockSpec((nb, WHALF), lambda h, i: (h * (NPAD // nb) + i, 0)),
            pl.BlockSpec((nb, WHALF), lambda h, i: (h * (NPAD // nb) + i, 0)),
            pl.BlockSpec((nb, WHALF), lambda h, i: (h * (NPAD // nb) + i, 0)),
            pl.BlockSpec((nb, 1), lambda h, i: (i, 0)),
        ],
        out_specs=pl.BlockSpec((nb, WHALF), lambda h, i: (h * (NPAD // nb) + i, 0)),
        out_shape=jax.ShapeDtypeStruct((2 * NPAD, WHALF), jnp.float32),
        name="cheb_combine",
    )(lam_arr, seg2, zp2, zpp2, deg2)


def _dense_body(z0_ref, z1_ref, z2_ref, x_ref, w0_ref, w1_ref, w2_ref,
                cb_ref, wt_ref, tb_ref, wr_ref, rb_ref, lg_ref, lb_ref,
                o_ref, *, nb):
    rows = nb * TB
    dot = functools.partial(jnp.dot, preferred_element_type=jnp.float32)
    r = (dot(z0_ref[...], w0_ref[...]) + dot(z1_ref[...], w1_ref[...])
         + dot(z2_ref[...], w2_ref[...]) + cb_ref[...])
    r = jnp.maximum(r, 0.0)
    rv = r.reshape(nb * B, T, F)
    zpad = jnp.zeros((nb * B, 1, F), jnp.float32)
    rp = jnp.concatenate([zpad, rv, zpad], axis=1)
    h = (dot(rp[:, 0:T, :].reshape(rows, F), wt_ref[0])
         + dot(rp[:, 1:T + 1, :].reshape(rows, F), wt_ref[1])
         + dot(rp[:, 2:T + 2, :].reshape(rows, F), wt_ref[2])
         + tb_ref[...])
    res = dot(x_ref[...], wr_ref[...]) + rb_ref[...]
    z = jnp.maximum(h + res, 0.0)
    mu = jnp.mean(z, axis=-1, keepdims=True)
    zc = z - mu
    var = jnp.mean(zc * zc, axis=-1, keepdims=True)
    o_ref[...] = zc * jax.lax.rsqrt(var + 1e-5) * lg_ref[...] + lb_ref[...]


def _dense(z0r, z1r, z2r, xrr, w0, w1, w2, cb, wt, tb, wr, rb, lg, lb):
    nb = 50
    rows = nb * TB
    grid = (N // nb,)
    row_spec = pl.BlockSpec((rows, C_IN), lambda i: (i, 0))
    full = lambda s: pl.BlockSpec(s, lambda i: tuple(0 for _ in s))
    return pl.pallas_call(
        functools.partial(_dense_body, nb=nb),
        grid=grid,
        in_specs=[row_spec, row_spec, row_spec, row_spec,
                  full((C_IN, F)), full((C_IN, F)), full((C_IN, F)),
                  full((1, F)), full((3, F, F)), full((1, F)),
                  full((C_IN, F)), full((1, F)), full((1, F)), full((1, F))],
        out_specs=pl.BlockSpec((rows, F), lambda i: (i, 0)),
        out_shape=jax.ShapeDtypeStruct((N * TB, F), jnp.float32),
        name="mstgcn_dense",
    )(z0r, z1r, z2r, xrr, w0, w1, w2, cb, wt, tb, wr, rb, lg, lb)


def kernel(X, edge_index, edge_weight, lambda_max, cheb_W, cheb_b,
           time_W, time_b, res_W, res_b, ln_g, ln_b):
    lam = jnp.asarray(lambda_max, jnp.float32)
    src = edge_index[0]
    dst = edge_index[1]
    pad = EPAD - E
    srcp = jnp.concatenate([src, jnp.zeros((pad,), jnp.int32)])
    dstp = jnp.concatenate([dst, jnp.zeros((pad,), jnp.int32)])
    wp = jnp.concatenate([edge_weight, jnp.zeros((pad,), jnp.float32)])
    scale_arr = jnp.full((16,), -2.0, jnp.float32) / lam

    Xt = jnp.transpose(X, (2, 0, 1, 3)).reshape(N, C_IN, TB)
    Xt = jnp.transpose(Xt, (2, 0, 1))
    xr = jnp.transpose(Xt, (1, 0, 2)).reshape(N, TB * C_IN)

    def to_split(z):
        zs = jnp.transpose(z.reshape(N, 2, HALF), (1, 0, 2))
        return jnp.pad(zs, ((0, 0), (0, NPAD - N), (0, WHALF - HALF)))

    def from_split2(z2):
        zs = z2.reshape(2, NPAD, WHALF)[:, :N, :HALF]
        return jnp.transpose(zs, (1, 0, 2)).reshape(N, 2 * HALF)

    _prop_sc = _make_prop_sc()
    z0s = to_split(xr)
    seg1, degp = _prop_sc(srcp, dstp, wp, z0s, scale_arr)
    deg2 = degp.reshape(NPAD, 1)
    lam_arr = lam.reshape(1, 1)

    z0f = z0s.reshape(2 * NPAD, WHALF)
    z1f = _combine(lam_arr, seg1.reshape(2 * NPAD, WHALF), z0f, z0f,
                   deg2, 1.0, 0.0)

    seg2, _ = _make_prop_sc(False)(srcp, dstp, wp,
                                   z1f.reshape(2, NPAD, WHALF), scale_arr)
    z2f = _combine(lam_arr, seg2.reshape(2 * NPAD, WHALF), z1f, z0f,
                   deg2, 2.0, -1.0)

    z0r = xr.reshape(N * TB, C_IN)
    z1r = from_split2(z1f).reshape(N * TB, C_IN)
    z2r = from_split2(z2f).reshape(N * TB, C_IN)
    xrr = jnp.transpose(X, (1, 0, 3, 2)).reshape(N * B * T, C_IN)

    w0, w1, w2 = cheb_W[0], cheb_W[1], cheb_W[2]
    cb = cheb_b.reshape(1, F)
    wt = jnp.transpose(time_W[:, :, 0, :], (2, 1, 0))
    tb = time_b.reshape(1, F)
    wr = jnp.transpose(res_W[:, :, 0, 0])
    rb = res_b.reshape(1, F)

    zout = _dense(z0r, z1r, z2r, xrr, w0, w1, w2, cb, wt, tb, wr, rb,
                  ln_g.reshape(1, F), ln_b.reshape(1, F))
    return jnp.transpose(zout.reshape(N, B, T, F), (1, 0, 3, 2))
